# Initial kernel scaffold; baseline (speedup 1.0000x reference)
import jax
import jax.numpy as jnp
from jax.experimental import pallas as pl


def kernel(scores, nBestIndex, werRank):
    raise NotImplementedError("write your pallas kernel here")


# speedup vs baseline: 3.7570x; 3.7570x over previous
import jax
import jax.numpy as jnp
from jax import lax
from jax.experimental import pallas as pl
from jax.experimental.pallas import tpu as pltpu
from jax.experimental.pallas import tpu_sc as plsc

_B = 16384
_N = 64
_MARGIN = 1.0
_NW = 32
_RPW = _B // _NW
_L = 16
_E = _RPW * _N


def _sc_body(scores_hbm, wr_hbm, out_hbm, chunk, wrv, partial):
    cid = lax.axis_index("c")
    sid = lax.axis_index("s")
    wid = sid * 2 + cid
    base = wid * _E

    pltpu.sync_copy(scores_hbm.at[pl.ds(base, _E)], chunk)
    pltpu.sync_copy(wr_hbm.at[pl.ds(base, _E)], wrv)

    def row_body(r, acc):
        off = r * _N
        vs = [chunk[pl.ds(off + j * _L, _L)] for j in range(_N // _L)]
        c0 = jnp.float32(_MARGIN) - vs[0][0]
        for v in vs:
            acc = acc + jnp.maximum(v + c0, jnp.float32(0.0))
        return acc

    acc = lax.fori_loop(0, _RPW, row_body, jnp.zeros((_L,), jnp.float32))

    def g_body(i, racc):
        row0 = (i * _L + lax.iota(jnp.int32, _L)) * _N
        r0 = plsc.load_gather(wrv, [row0])
        posv = plsc.load_gather(chunk, [row0])
        g = plsc.load_gather(chunk, [row0 + r0])
        return racc + jnp.maximum(g - posv + jnp.float32(_MARGIN), jnp.float32(0.0))

    racc = lax.fori_loop(0, _RPW // _L, g_body, jnp.zeros((_L,), jnp.float32))

    partial[...] = (acc - racc) * jnp.float32(1.0 / (_N - 1))
    pltpu.sync_copy(partial, out_hbm.at[wid])


def kernel(scores, nBestIndex, werRank):
    wr_flat = werRank.reshape(_B * _N)
    mesh = plsc.VectorSubcoreMesh(core_axis_name="c", subcore_axis_name="s")
    out = pl.kernel(
        _sc_body,
        mesh=mesh,
        out_type=jax.ShapeDtypeStruct((_NW, _L), jnp.float32),
        scratch_types=[
            pltpu.VMEM((_E,), jnp.float32),
            pltpu.VMEM((_E,), jnp.int32),
            pltpu.VMEM((_L,), jnp.float32),
        ],
        compiler_params=pltpu.CompilerParams(needs_layout_passes=False),
    )(scores, wr_flat)
    return jnp.sum(out).reshape(1)


# speedup vs baseline: 4.0288x; 1.0724x over previous
import jax
import jax.numpy as jnp
from jax import lax
from jax.experimental import pallas as pl
from jax.experimental.pallas import tpu as pltpu
from jax.experimental.pallas import tpu_sc as plsc

_B = 16384
_N = 64
_MARGIN = 1.0
_NW = 32
_RPW = _B // _NW
_L = 16
_E = _RPW * _N
_NSLAB = 4
_RSLAB = _RPW // _NSLAB
_ESLAB = _RSLAB * _N


def _sc_body(scores_hbm, wr_hbm, out_hbm, chunk, idx, wr0, partial,
             sems, semw):
    cid = lax.axis_index("c")
    sid = lax.axis_index("s")
    wid = sid * 2 + cid
    base_row = wid * _RPW
    iota = lax.iota(jnp.int32, _L)

    copies = []
    for k in range(_NSLAB):
        copies.append(pltpu.async_copy(
            scores_hbm.at[pl.ds(wid * _E + k * _ESLAB, _ESLAB)],
            chunk.at[pl.ds(k * _ESLAB, _ESLAB)], sems[k]))

    for k in range(_NSLAB):
        for m in range(_RSLAB // _L):
            rows = base_row + k * _RSLAB + m * _L + iota
            idx[k, pl.ds(m * _L, _L)] = rows * _N
    wr_copies = [
        pltpu.async_copy(wr_hbm.at[idx.at[k]], wr0.at[k], semw)
        for k in range(_NSLAB)
    ]

    accs = (jnp.zeros((_L,), jnp.float32),) * 4

    def row_body(r, accs):
        off = r * _N
        posplat = plsc.load_gather(chunk, [jnp.full((_L,), off, jnp.int32)])
        c0 = jnp.float32(_MARGIN) - posplat
        new = []
        for j in range(_N // _L):
            v = chunk[pl.ds(off + j * _L, _L)]
            new.append(accs[j] + jnp.maximum(v + c0, jnp.float32(0.0)))
        return tuple(new)

    for k in range(_NSLAB):
        copies[k].wait()
        accs = lax.fori_loop(k * _RSLAB, (k + 1) * _RSLAB, row_body, accs,
                             unroll=2)

    for c in wr_copies:
        c.wait()
    racc = jnp.zeros((_L,), jnp.float32)
    for k in range(_NSLAB):
        for m in range(_RSLAB // _L):
            loff = ((k * _RSLAB + m * _L) + iota) * _N
            r0 = wr0[k, pl.ds(m * _L, _L)]
            posv = plsc.load_gather(chunk, [loff])
            g = plsc.load_gather(chunk, [loff + r0])
            racc = racc + jnp.maximum(g - posv + jnp.float32(_MARGIN),
                                      jnp.float32(0.0))

    total = accs[0] + accs[1] + accs[2] + accs[3] - racc
    partial[...] = total * jnp.float32(1.0 / (_N - 1))
    pltpu.sync_copy(partial, out_hbm.at[wid])


def kernel(scores, nBestIndex, werRank):
    wr_flat = werRank.reshape(_B * _N)
    mesh = plsc.VectorSubcoreMesh(core_axis_name="c", subcore_axis_name="s")
    out = pl.kernel(
        _sc_body,
        mesh=mesh,
        out_type=jax.ShapeDtypeStruct((_NW, _L), jnp.float32),
        scratch_types=[
            pltpu.VMEM((_E,), jnp.float32),
            pltpu.VMEM((_NSLAB, _RSLAB), jnp.int32),
            pltpu.VMEM((_NSLAB, _RSLAB), jnp.int32),
            pltpu.VMEM((_L,), jnp.float32),
            [pltpu.SemaphoreType.DMA] * _NSLAB,
            pltpu.SemaphoreType.DMA,
        ],
        compiler_params=pltpu.CompilerParams(needs_layout_passes=False),
    )(scores, wr_flat)
    return jnp.sum(out).reshape(1)
